# baseline (speedup 1.0000x reference)
import functools

import jax
import jax.numpy as jnp
from jax import lax
from jax.experimental import pallas as pl
from jax.experimental.pallas import tpu as pltpu
from jax.experimental.pallas import tpu_sc as plsc

DIM = 128
NC = 2
NS = 16
NW = NC * NS
NBUF = 5
LOOKAHEAD = 3


@functools.lru_cache(maxsize=None)
def _make_kernel(B, S):
    b_per_w = B // NW
    assert S % NBUF == 0
    mesh = plsc.VectorSubcoreMesh(core_axis_name="c", subcore_axis_name="s")

    @functools.partial(
        pl.kernel,
        out_type=jax.ShapeDtypeStruct((S, B, DIM), jnp.float32),
        mesh=mesh,
        scratch_types=[
            pltpu.VMEM((S, 1, b_per_w), jnp.int32),
            [pltpu.VMEM((b_per_w, DIM), jnp.float32) for _ in range(NBUF)],
            [pltpu.SemaphoreType.DMA for _ in range(NBUF)],
            [pltpu.SemaphoreType.DMA for _ in range(NBUF)],
        ],
    )
    def body(idx_hbm, table_hbm, out_hbm, idx_v, bufs, gsems, osems):
        wid = lax.axis_index("s") * NC + lax.axis_index("c")
        b0 = wid * b_per_w
        pltpu.sync_copy(idx_hbm.at[:, pl.ds(wid, 1)], idx_v)

        for k in range(LOOKAHEAD):
            pltpu.async_copy(table_hbm.at[idx_v.at[k, 0]], bufs[k], gsems[k])

        def group(g, carry):
            for k in range(NBUF):
                s = g * NBUF + k
                pltpu.make_async_copy(
                    table_hbm.at[idx_v.at[s, 0]], bufs[k], gsems[k]).wait()
                pltpu.async_copy(
                    bufs[k], out_hbm.at[s, pl.ds(b0, b_per_w)], osems[k])

                t = s + LOOKAHEAD
                kt = (k + LOOKAHEAD) % NBUF

                @pl.when(t < S)
                def _():
                    @pl.when(t >= NBUF)
                    def _():
                        pltpu.make_async_copy(
                            bufs[kt],
                            out_hbm.at[t - NBUF, pl.ds(b0, b_per_w)],
                            osems[kt]).wait()

                    pltpu.async_copy(
                        table_hbm.at[idx_v.at[t, 0]], bufs[kt], gsems[kt])

            return carry

        lax.fori_loop(0, S // NBUF, group, 0)

        for j in range(S - NBUF + LOOKAHEAD, S):
            pltpu.make_async_copy(
                bufs[j % NBUF], out_hbm.at[j, pl.ds(b0, b_per_w)],
                osems[j % NBUF]).wait()

    return body


def kernel(x_word, table):
    B, S = x_word.shape
    b_per_w = B // NW
    idx = x_word.astype(jnp.int32).T.reshape(S, NW, b_per_w)
    out = _make_kernel(B, S)(idx, table)
    return out.transpose(1, 0, 2)


# speedup vs baseline: 1.0048x; 1.0048x over previous
import functools

import jax
import jax.numpy as jnp
from jax import lax
from jax.experimental import pallas as pl
from jax.experimental.pallas import tpu as pltpu
from jax.experimental.pallas import tpu_sc as plsc

DIM = 128
NC = 2
NS = 16
NW = NC * NS
NBUF = 5


@functools.lru_cache(maxsize=None)
def _make_kernel(B, S):
    b_per_w = B // NW
    assert S % NBUF == 0
    mesh = plsc.VectorSubcoreMesh(core_axis_name="c", subcore_axis_name="s")

    @functools.partial(
        pl.kernel,
        out_type=jax.ShapeDtypeStruct((S, B, DIM), jnp.float32),
        mesh=mesh,
        scratch_types=[
            pltpu.VMEM((S, 1, b_per_w), jnp.int32),
            [pltpu.VMEM((b_per_w, DIM), jnp.float32) for _ in range(NBUF)],
            [pltpu.SemaphoreType.DMA for _ in range(NBUF)],
            [pltpu.SemaphoreType.DMA for _ in range(NBUF)],
        ],
    )
    def body(idx_hbm, table_hbm, out_hbm, idx_v, bufs, gsems, osems):
        wid = lax.axis_index("s") * NC + lax.axis_index("c")
        b0 = wid * b_per_w
        pltpu.sync_copy(idx_hbm.at[:, pl.ds(wid, 1)], idx_v)

        for k in range(NBUF):
            pltpu.async_copy(table_hbm.at[idx_v.at[k, 0]], bufs[k], gsems[k])

        def group(g, carry):
            for k in range(NBUF):
                s = g * NBUF + k
                pltpu.make_async_copy(
                    table_hbm.at[idx_v.at[s, 0]], bufs[k], gsems[k]).wait()
                pltpu.async_copy(
                    bufs[k], out_hbm.at[s, pl.ds(b0, b_per_w)], osems[k])
                nxt = s + NBUF

                @pl.when(nxt < S)
                def _():
                    pltpu.make_async_copy(
                        bufs[k], out_hbm.at[s, pl.ds(b0, b_per_w)],
                        osems[k]).wait()
                    pltpu.async_copy(
                        table_hbm.at[idx_v.at[nxt, 0]], bufs[k], gsems[k])

            return carry

        lax.fori_loop(0, S // NBUF, group, 0)

        for k in range(NBUF):
            s = S - NBUF + k
            pltpu.make_async_copy(
                bufs[k], out_hbm.at[s, pl.ds(b0, b_per_w)], osems[k]).wait()

    return body


def kernel(x_word, table):
    B, S = x_word.shape
    b_per_w = B // NW
    idx = x_word.astype(jnp.int32).T.reshape(S, NW, b_per_w)
    out = _make_kernel(B, S)(idx, table)
    return out.transpose(1, 0, 2)
